# baseline (speedup 1.0000x reference)
import functools

import jax
import jax.numpy as jnp
from jax import lax
from jax.experimental import pallas as pl
from jax.experimental.pallas import tpu as pltpu
from jax.experimental.pallas import tpu_sc as plsc


def _leaky(x):
    return jnp.where(x >= 0, x, 0.01 * x)


_RB = 1000
_CH = 128
_NC = 2
_NS = 16


def _fusion_body(des_ref, tw_ref, np_ref, cp_ref,
                 wd_ref, bd_ref, wt_ref, bt_ref, wn_ref, bn_ref,
                 wc_ref, bc_ref, wi_ref, bi_ref, o_ref):
    d = _leaky(jnp.dot(des_ref[...], wd_ref[...],
                       preferred_element_type=jnp.float32) + bd_ref[...])
    t = _leaky(jnp.dot(tw_ref[...], wt_ref[...],
                       preferred_element_type=jnp.float32) + bt_ref[...])
    n = _leaky(jnp.dot(np_ref[...], wn_ref[...],
                       preferred_element_type=jnp.float32) + bn_ref[...])
    c = _leaky(jnp.dot(cp_ref[...], wc_ref[...],
                       preferred_element_type=jnp.float32) + bc_ref[...])
    x = jnp.concatenate([d, t, n, c], axis=1)
    o_ref[...] = _leaky(jnp.dot(x, wi_ref[...],
                                preferred_element_type=jnp.float32) + bi_ref[...])


def _fusion(des, tweet, num_prop, cat_prop,
            W_des, b_des, W_tw, b_tw, W_np, b_np, W_cp, b_cp, W_in, b_in):
    n = des.shape[0]
    q = W_des.shape[1]
    d = W_in.shape[1]
    row = lambda i: (i, 0)
    full = lambda i: (0, 0)
    return pl.pallas_call(
        _fusion_body,
        grid=(n // _RB,),
        in_specs=[
            pl.BlockSpec((_RB, des.shape[1]), row),
            pl.BlockSpec((_RB, tweet.shape[1]), row),
            pl.BlockSpec((_RB, num_prop.shape[1]), row),
            pl.BlockSpec((_RB, cat_prop.shape[1]), row),
            pl.BlockSpec(W_des.shape, full), pl.BlockSpec((1, q), full),
            pl.BlockSpec(W_tw.shape, full), pl.BlockSpec((1, q), full),
            pl.BlockSpec(W_np.shape, full), pl.BlockSpec((1, q), full),
            pl.BlockSpec(W_cp.shape, full), pl.BlockSpec((1, q), full),
            pl.BlockSpec(W_in.shape, full), pl.BlockSpec((1, d), full),
        ],
        out_specs=pl.BlockSpec((_RB, d), row),
        out_shape=jax.ShapeDtypeStruct((n, d), jnp.float32),
    )(des, tweet, num_prop, cat_prop,
      W_des, b_des.reshape(1, -1), W_tw, b_tw.reshape(1, -1),
      W_np, b_np.reshape(1, -1), W_cp, b_cp.reshape(1, -1),
      W_in, b_in.reshape(1, -1))


def _dinv_of(degp):
    return lax.rsqrt(degp[0] + degp[1] + 1.0)


def _pre_body(x_ref, w_ref, degp_ref, o_ref):
    h = jnp.dot(x_ref[...], w_ref[...], preferred_element_type=jnp.float32)
    g = _dinv_of(degp_ref[...]) * h
    d2 = g.shape[1] // 2
    o_ref[0] = g[:, :d2]
    o_ref[1] = g[:, d2:]


def _mid_body(s_ref, g_ref, degp_ref, b_ref, w_ref, o_ref):
    dinv = _dinv_of(degp_ref[...])
    s = jnp.concatenate([s_ref[0], s_ref[1]], axis=1)
    g = jnp.concatenate([g_ref[0], g_ref[1]], axis=1)
    x1 = dinv * (s + g) + b_ref[...]
    g2 = dinv * jnp.dot(x1, w_ref[...], preferred_element_type=jnp.float32)
    d2 = g2.shape[1] // 2
    o_ref[0] = g2[:, :d2]
    o_ref[1] = g2[:, d2:]


def _post_body(s_ref, g_ref, degp_ref, b_ref, w1_ref, b1_ref, w2_ref, b2_ref,
               o_ref):
    dinv = _dinv_of(degp_ref[...])
    s = jnp.concatenate([s_ref[0], s_ref[1]], axis=1)
    g = jnp.concatenate([g_ref[0], g_ref[1]], axis=1)
    x2 = dinv * (s + g) + b_ref[...]
    h = _leaky(jnp.dot(x2, w1_ref[...],
                       preferred_element_type=jnp.float32) + b1_ref[...])
    o_ref[...] = jnp.dot(h, w2_ref[...],
                         preferred_element_type=jnp.float32) + b2_ref[...]


def _sc_meshes():
    return plsc.VectorSubcoreMesh(core_axis_name="c", subcore_axis_name="s")


def _deg_call(dst2, nacc):
    rows = dst2.shape[0]
    rpt = rows // (_NC * _NS)
    grp = 4
    zcp = nacc // _NS // _CH

    @functools.partial(
        pl.kernel,
        out_type=jax.ShapeDtypeStruct((_NC, nacc), jnp.float32),
        mesh=_sc_meshes(),
        compiler_params=pltpu.CompilerParams(use_tc_tiling_on_sc=False),
        scratch_types=[
            pltpu.VMEM_SHARED((nacc,), jnp.float32),
            pltpu.VMEM((grp, _CH), jnp.int32),
            pltpu.VMEM((_CH,), jnp.float32),
            pltpu.VMEM((_CH,), jnp.float32),
        ],
    )
    def deg_kernel(dst2_hbm, degp_hbm, acc, idxb, onesb, zb):
        c = lax.axis_index("c")
        s = lax.axis_index("s")

        def fill(i, _):
            onesb[pl.ds(i * 16, 16)] = jnp.full((16,), 1.0, jnp.float32)
            zb[pl.ds(i * 16, 16)] = jnp.zeros((16,), jnp.float32)
            return 0
        lax.fori_loop(0, _CH // 16, fill, 0)

        def zero(i, _):
            pltpu.sync_copy(zb, acc.at[pl.ds((s * zcp + i) * _CH, _CH)])
            return 0
        lax.fori_loop(0, zcp, zero, 0)
        plsc.subcore_barrier()

        base = (c * _NS + s) * rpt

        def body(gi, _):
            row0 = base + gi * grp
            pltpu.sync_copy(dst2_hbm.at[pl.ds(row0, grp)], idxb)
            for j in range(grp):
                pltpu.sync_copy(onesb, acc.at[idxb.at[j]], add=True)
            return 0
        lax.fori_loop(0, rpt // grp, body, 0)
        plsc.subcore_barrier()

        def out(i, _):
            off = (s * zcp + i) * _CH
            pltpu.sync_copy(acc.at[pl.ds(off, _CH)],
                            degp_hbm.at[c, pl.ds(off, _CH)])
            return 0
        lax.fori_loop(0, zcp, out, 0)

    return deg_kernel(dst2)


def _agg_call(csr, gflat, nacc, rows):
    d2 = gflat.shape[1]
    rpt = rows // _NS
    grp = 5
    ngrp = rpt // grp
    zcp = nacc // _NS // _CH

    @functools.partial(
        pl.kernel,
        out_type=jax.ShapeDtypeStruct((_NC, nacc, d2), jnp.float32),
        mesh=_sc_meshes(),
        compiler_params=pltpu.CompilerParams(use_tc_tiling_on_sc=False),
        scratch_types=[
            pltpu.VMEM_SHARED((nacc, d2), jnp.float32),
            pltpu.VMEM((2, 2 * grp, _CH), jnp.int32),
            pltpu.VMEM((grp, _CH, d2), jnp.float32),
            pltpu.SemaphoreType.DMA,
            pltpu.SemaphoreType.DMA,
        ],
    )
    def agg_kernel(csr_hbm, g_hbm, sout_hbm, acc, cbuf, rbuf, gsem, isem):
        c = lax.axis_index("c")
        s = lax.axis_index("s")

        def zfill(i, _):
            for k in range(d2 // 16):
                rbuf[0, i, pl.ds(k * 16, 16)] = jnp.zeros((16,), jnp.float32)
            return 0
        lax.fori_loop(0, _CH, zfill, 0)

        def zero(i, _):
            pltpu.sync_copy(rbuf.at[0], acc.at[pl.ds((s * zcp + i) * _CH, _CH)])
            return 0
        lax.fori_loop(0, zcp, zero, 0)
        plsc.subcore_barrier()

        base2 = s * 2 * rpt
        pltpu.sync_copy(csr_hbm.at[c, pl.ds(base2, 2 * grp)], cbuf.at[0])

        def body(g, _):
            b = g % 2

            @pl.when(g < ngrp - 1)
            def _stage_next():
                r0 = base2 + (g + 1) * 2 * grp
                pltpu.async_copy(csr_hbm.at[c, pl.ds(r0, 2 * grp)],
                                 cbuf.at[1 - b], isem)

            for j in range(grp):
                pltpu.async_copy(g_hbm.at[cbuf.at[b, 2 * j]], rbuf.at[j],
                                 gsem)
            for j in range(grp):
                pltpu.make_async_copy(g_hbm.at[cbuf.at[b, 2 * j]],
                                      rbuf.at[j], gsem).wait()
            for j in range(grp):
                pltpu.sync_copy(rbuf.at[j], acc.at[cbuf.at[b, 2 * j + 1]],
                                add=True)

            @pl.when(g < ngrp - 1)
            def _wait_next():
                r0 = base2 + (g + 1) * 2 * grp
                pltpu.make_async_copy(csr_hbm.at[c, pl.ds(r0, 2 * grp)],
                                      cbuf.at[1 - b], isem).wait()
            return 0
        lax.fori_loop(0, ngrp, body, 0)
        plsc.subcore_barrier()

        def out(i, _):
            off = (s * zcp + i) * _CH
            pltpu.sync_copy(acc.at[pl.ds(off, _CH)],
                            sout_hbm.at[c, pl.ds(off, _CH)])
            return 0
        lax.fori_loop(0, zcp, out, 0)

    return agg_kernel(csr, gflat)


def kernel(des, tweet, num_prop, cat_prop, edge_index, edge_type,
           W_des, b_des, W_tw, b_tw, W_np, b_np, W_cp, b_cp,
           W_in, b_in, Wg1, bg1, Wg2, bg2, W_o1, b_o1, W_o2, b_o2):
    n = des.shape[0]
    e = edge_index.shape[1]
    d = W_in.shape[1]
    d2 = d // 2

    unit = _CH * _NS * 16
    pe = ((e + unit - 1) // unit) * unit
    pad = pe - e
    garbage = 1200
    nacc = ((n + garbage + _NS * _CH - 1) // (_NS * _CH)) * (_NS * _CH)
    src = jnp.concatenate(
        [edge_index[0], jnp.zeros((pad,), edge_index.dtype)])
    dst = jnp.concatenate(
        [edge_index[1],
         n + (jnp.arange(pad, dtype=edge_index.dtype) % garbage)])
    rows = pe // _CH
    src2 = src.reshape(rows, _CH)
    dst2 = dst.reshape(rows, _CH)
    csr = jnp.stack([
        jnp.stack([src2, dst2], axis=1).reshape(2 * rows, _CH),
        jnp.stack([src2 + n, dst2], axis=1).reshape(2 * rows, _CH),
    ])

    degp = _deg_call(dst2, nacc).reshape(_NC, nacc, 1)

    x = _fusion(des, tweet, num_prop, cat_prop,
                W_des, b_des, W_tw, b_tw, W_np, b_np, W_cp, b_cp, W_in, b_in)

    row = lambda i: (i, 0)
    full = lambda i: (0, 0)
    stk = lambda i: (0, i, 0)
    spec_half = pl.BlockSpec((_NC, _RB, d2), stk)
    spec_deg = pl.BlockSpec((_NC, _RB, 1), stk)
    grid = (n // _RB,)

    g1 = pl.pallas_call(
        _pre_body,
        grid=grid,
        in_specs=[pl.BlockSpec((_RB, d), row), pl.BlockSpec(Wg1.shape, full),
                  spec_deg],
        out_specs=spec_half,
        out_shape=jax.ShapeDtypeStruct((_NC, n, d2), jnp.float32),
    )(x, Wg1, degp)

    s1 = _agg_call(csr, g1.reshape(_NC * n, d2), nacc, rows)

    g2 = pl.pallas_call(
        _mid_body,
        grid=grid,
        in_specs=[spec_half, spec_half, spec_deg,
                  pl.BlockSpec((1, d), full), pl.BlockSpec(Wg2.shape, full)],
        out_specs=spec_half,
        out_shape=jax.ShapeDtypeStruct((_NC, n, d2), jnp.float32),
    )(s1, g1, degp, bg1.reshape(1, -1), Wg2)

    s2 = _agg_call(csr, g2.reshape(_NC * n, d2), nacc, rows)

    out = pl.pallas_call(
        _post_body,
        grid=grid,
        in_specs=[spec_half, spec_half, spec_deg,
                  pl.BlockSpec((1, d), full),
                  pl.BlockSpec(W_o1.shape, full), pl.BlockSpec((1, d), full),
                  pl.BlockSpec(W_o2.shape, full),
                  pl.BlockSpec((1, W_o2.shape[1]), full)],
        out_specs=pl.BlockSpec((_RB, W_o2.shape[1]), row),
        out_shape=jax.ShapeDtypeStruct((n, W_o2.shape[1]), jnp.float32),
    )(s2, g2, degp, bg2.reshape(1, -1),
      W_o1, b_o1.reshape(1, -1), W_o2, b_o2.reshape(1, -1))

    return out


# speedup vs baseline: 1.0039x; 1.0039x over previous
import functools

import jax
import jax.numpy as jnp
from jax import lax
from jax.experimental import pallas as pl
from jax.experimental.pallas import tpu as pltpu
from jax.experimental.pallas import tpu_sc as plsc


def _leaky(x):
    return jnp.where(x >= 0, x, 0.01 * x)


_RB = 1000
_CH = 128
_NC = 2
_NS = 16


def _fusion_body(des_ref, tw_ref, np_ref, cp_ref,
                 wd_ref, bd_ref, wt_ref, bt_ref, wn_ref, bn_ref,
                 wc_ref, bc_ref, wi_ref, bi_ref, o_ref):
    d = _leaky(jnp.dot(des_ref[...], wd_ref[...],
                       preferred_element_type=jnp.float32) + bd_ref[...])
    t = _leaky(jnp.dot(tw_ref[...], wt_ref[...],
                       preferred_element_type=jnp.float32) + bt_ref[...])
    n = _leaky(jnp.dot(np_ref[...], wn_ref[...],
                       preferred_element_type=jnp.float32) + bn_ref[...])
    c = _leaky(jnp.dot(cp_ref[...], wc_ref[...],
                       preferred_element_type=jnp.float32) + bc_ref[...])
    x = jnp.concatenate([d, t, n, c], axis=1)
    o_ref[...] = _leaky(jnp.dot(x, wi_ref[...],
                                preferred_element_type=jnp.float32) + bi_ref[...])


def _fusion(des, tweet, num_prop, cat_prop,
            W_des, b_des, W_tw, b_tw, W_np, b_np, W_cp, b_cp, W_in, b_in):
    n = des.shape[0]
    q = W_des.shape[1]
    d = W_in.shape[1]
    row = lambda i: (i, 0)
    full = lambda i: (0, 0)
    return pl.pallas_call(
        _fusion_body,
        grid=(n // _RB,),
        in_specs=[
            pl.BlockSpec((_RB, des.shape[1]), row),
            pl.BlockSpec((_RB, tweet.shape[1]), row),
            pl.BlockSpec((_RB, num_prop.shape[1]), row),
            pl.BlockSpec((_RB, cat_prop.shape[1]), row),
            pl.BlockSpec(W_des.shape, full), pl.BlockSpec((1, q), full),
            pl.BlockSpec(W_tw.shape, full), pl.BlockSpec((1, q), full),
            pl.BlockSpec(W_np.shape, full), pl.BlockSpec((1, q), full),
            pl.BlockSpec(W_cp.shape, full), pl.BlockSpec((1, q), full),
            pl.BlockSpec(W_in.shape, full), pl.BlockSpec((1, d), full),
        ],
        out_specs=pl.BlockSpec((_RB, d), row),
        out_shape=jax.ShapeDtypeStruct((n, d), jnp.float32),
    )(des, tweet, num_prop, cat_prop,
      W_des, b_des.reshape(1, -1), W_tw, b_tw.reshape(1, -1),
      W_np, b_np.reshape(1, -1), W_cp, b_cp.reshape(1, -1),
      W_in, b_in.reshape(1, -1))


def _dinv_of(degp):
    return lax.rsqrt(degp[0] + degp[1] + 1.0)


def _pre_body(x_ref, w_ref, degp_ref, o_ref):
    h = jnp.dot(x_ref[...], w_ref[...], preferred_element_type=jnp.float32)
    g = _dinv_of(degp_ref[...]) * h
    d2 = g.shape[1] // 2
    o_ref[0] = g[:, :d2]
    o_ref[1] = g[:, d2:]


def _mid_body(s_ref, g_ref, degp_ref, b_ref, w_ref, o_ref):
    dinv = _dinv_of(degp_ref[...])
    s = jnp.concatenate([s_ref[0], s_ref[1]], axis=1)
    g = jnp.concatenate([g_ref[0], g_ref[1]], axis=1)
    x1 = dinv * (s + g) + b_ref[...]
    g2 = dinv * jnp.dot(x1, w_ref[...], preferred_element_type=jnp.float32)
    d2 = g2.shape[1] // 2
    o_ref[0] = g2[:, :d2]
    o_ref[1] = g2[:, d2:]


def _post_body(s_ref, g_ref, degp_ref, b_ref, w1_ref, b1_ref, w2_ref, b2_ref,
               o_ref):
    dinv = _dinv_of(degp_ref[...])
    s = jnp.concatenate([s_ref[0], s_ref[1]], axis=1)
    g = jnp.concatenate([g_ref[0], g_ref[1]], axis=1)
    x2 = dinv * (s + g) + b_ref[...]
    h = _leaky(jnp.dot(x2, w1_ref[...],
                       preferred_element_type=jnp.float32) + b1_ref[...])
    o_ref[...] = jnp.dot(h, w2_ref[...],
                         preferred_element_type=jnp.float32) + b2_ref[...]


def _sc_meshes():
    return plsc.VectorSubcoreMesh(core_axis_name="c", subcore_axis_name="s")


def _deg_call(dst2, nacc):
    rows = dst2.shape[0]
    rpt = rows // (_NC * _NS)
    grp = 4
    zcp = nacc // _NS // _CH

    @functools.partial(
        pl.kernel,
        out_type=jax.ShapeDtypeStruct((_NC, nacc), jnp.float32),
        mesh=_sc_meshes(),
        compiler_params=pltpu.CompilerParams(use_tc_tiling_on_sc=False),
        scratch_types=[
            pltpu.VMEM_SHARED((nacc,), jnp.float32),
            pltpu.VMEM((grp, _CH), jnp.int32),
            pltpu.VMEM((_CH,), jnp.float32),
            pltpu.VMEM((_CH,), jnp.float32),
        ],
    )
    def deg_kernel(dst2_hbm, degp_hbm, acc, idxb, onesb, zb):
        c = lax.axis_index("c")
        s = lax.axis_index("s")

        def fill(i, _):
            onesb[pl.ds(i * 16, 16)] = jnp.full((16,), 1.0, jnp.float32)
            zb[pl.ds(i * 16, 16)] = jnp.zeros((16,), jnp.float32)
            return 0
        lax.fori_loop(0, _CH // 16, fill, 0)

        def zero(i, _):
            pltpu.sync_copy(zb, acc.at[pl.ds((s * zcp + i) * _CH, _CH)])
            return 0
        lax.fori_loop(0, zcp, zero, 0)
        plsc.subcore_barrier()

        base = (c * _NS + s) * rpt

        def body(gi, _):
            row0 = base + gi * grp
            pltpu.sync_copy(dst2_hbm.at[pl.ds(row0, grp)], idxb)
            for j in range(grp):
                pltpu.sync_copy(onesb, acc.at[idxb.at[j]], add=True)
            return 0
        lax.fori_loop(0, rpt // grp, body, 0)
        plsc.subcore_barrier()

        def out(i, _):
            off = (s * zcp + i) * _CH
            pltpu.sync_copy(acc.at[pl.ds(off, _CH)],
                            degp_hbm.at[c, pl.ds(off, _CH)])
            return 0
        lax.fori_loop(0, zcp, out, 0)

    return deg_kernel(dst2)


def _agg_call(csr, gflat, nacc, rows):
    d2 = gflat.shape[1]
    rpt = rows // _NS
    grp = 5
    ngrp = rpt // grp
    zcp = nacc // _NS // _CH

    @functools.partial(
        pl.kernel,
        out_type=jax.ShapeDtypeStruct((_NC, nacc, d2), jnp.float32),
        mesh=_sc_meshes(),
        compiler_params=pltpu.CompilerParams(use_tc_tiling_on_sc=False),
        scratch_types=[
            pltpu.VMEM_SHARED((nacc, d2), jnp.float32),
            pltpu.VMEM((2, 2 * grp, _CH), jnp.int32),
            pltpu.VMEM((grp, _CH, d2), jnp.float32),
            pltpu.SemaphoreType.DMA,
            pltpu.SemaphoreType.DMA,
        ],
    )
    def agg_kernel(csr_hbm, g_hbm, sout_hbm, acc, cbuf, rbuf, gsem, isem):
        c = lax.axis_index("c")
        s = lax.axis_index("s")

        def zfill(i, _):
            for k in range(d2 // 16):
                rbuf[0, i, pl.ds(k * 16, 16)] = jnp.zeros((16,), jnp.float32)
            return 0
        lax.fori_loop(0, _CH, zfill, 0)

        def zero(i, _):
            pltpu.sync_copy(rbuf.at[0], acc.at[pl.ds((s * zcp + i) * _CH, _CH)])
            return 0
        lax.fori_loop(0, zcp, zero, 0)
        plsc.subcore_barrier()

        base2 = s * 2 * rpt
        pltpu.sync_copy(csr_hbm.at[c, pl.ds(base2, 2 * grp)], cbuf.at[0])

        def proc(buf):
            cps = [pltpu.async_copy(g_hbm.at[buf.at[2 * j]], rbuf.at[j],
                                    gsem) for j in range(grp)]
            for cp in cps:
                cp.wait()
            for j in range(grp):
                pltpu.sync_copy(rbuf.at[j], acc.at[buf.at[2 * j + 1]],
                                add=True)

        def body(t, _):
            r1 = base2 + (2 * t + 1) * 2 * grp
            d1 = pltpu.async_copy(csr_hbm.at[c, pl.ds(r1, 2 * grp)],
                                  cbuf.at[1], isem)
            proc(cbuf.at[0])
            d1.wait()
            r2 = base2 + (2 * t + 2) * 2 * grp
            d2 = pltpu.async_copy(csr_hbm.at[c, pl.ds(r2, 2 * grp)],
                                  cbuf.at[0], isem)
            proc(cbuf.at[1])
            d2.wait()
            return 0
        lax.fori_loop(0, ngrp // 2, body, 0)
        plsc.subcore_barrier()

        def out(i, _):
            off = (s * zcp + i) * _CH
            pltpu.sync_copy(acc.at[pl.ds(off, _CH)],
                            sout_hbm.at[c, pl.ds(off, _CH)])
            return 0
        lax.fori_loop(0, zcp, out, 0)

    return agg_kernel(csr, gflat)


def kernel(des, tweet, num_prop, cat_prop, edge_index, edge_type,
           W_des, b_des, W_tw, b_tw, W_np, b_np, W_cp, b_cp,
           W_in, b_in, Wg1, bg1, Wg2, bg2, W_o1, b_o1, W_o2, b_o2):
    n = des.shape[0]
    e = edge_index.shape[1]
    d = W_in.shape[1]
    d2 = d // 2

    unit = _CH * _NS * 16
    pe = ((e + unit - 1) // unit) * unit
    pad = pe - e
    garbage = 1200
    nacc = ((n + garbage + _NS * _CH - 1) // (_NS * _CH)) * (_NS * _CH)
    src = jnp.concatenate(
        [edge_index[0], jnp.zeros((pad,), edge_index.dtype)])
    dst = jnp.concatenate(
        [edge_index[1],
         n + (jnp.arange(pad, dtype=edge_index.dtype) % garbage)])
    rows = pe // _CH
    src2 = src.reshape(rows, _CH)
    dst2 = dst.reshape(rows, _CH)
    zpad = jnp.zeros((16, _CH), jnp.int32)
    csr = jnp.stack([
        jnp.concatenate(
            [jnp.stack([src2, dst2], axis=1).reshape(2 * rows, _CH), zpad]),
        jnp.concatenate(
            [jnp.stack([src2 + n, dst2], axis=1).reshape(2 * rows, _CH),
             zpad]),
    ])

    degp = _deg_call(dst2, nacc).reshape(_NC, nacc, 1)

    x = _fusion(des, tweet, num_prop, cat_prop,
                W_des, b_des, W_tw, b_tw, W_np, b_np, W_cp, b_cp, W_in, b_in)

    row = lambda i: (i, 0)
    full = lambda i: (0, 0)
    stk = lambda i: (0, i, 0)
    spec_half = pl.BlockSpec((_NC, _RB, d2), stk)
    spec_deg = pl.BlockSpec((_NC, _RB, 1), stk)
    grid = (n // _RB,)

    g1 = pl.pallas_call(
        _pre_body,
        grid=grid,
        in_specs=[pl.BlockSpec((_RB, d), row), pl.BlockSpec(Wg1.shape, full),
                  spec_deg],
        out_specs=spec_half,
        out_shape=jax.ShapeDtypeStruct((_NC, n, d2), jnp.float32),
    )(x, Wg1, degp)

    s1 = _agg_call(csr, g1.reshape(_NC * n, d2), nacc, rows)

    g2 = pl.pallas_call(
        _mid_body,
        grid=grid,
        in_specs=[spec_half, spec_half, spec_deg,
                  pl.BlockSpec((1, d), full), pl.BlockSpec(Wg2.shape, full)],
        out_specs=spec_half,
        out_shape=jax.ShapeDtypeStruct((_NC, n, d2), jnp.float32),
    )(s1, g1, degp, bg1.reshape(1, -1), Wg2)

    s2 = _agg_call(csr, g2.reshape(_NC * n, d2), nacc, rows)

    out = pl.pallas_call(
        _post_body,
        grid=grid,
        in_specs=[spec_half, spec_half, spec_deg,
                  pl.BlockSpec((1, d), full),
                  pl.BlockSpec(W_o1.shape, full), pl.BlockSpec((1, d), full),
                  pl.BlockSpec(W_o2.shape, full),
                  pl.BlockSpec((1, W_o2.shape[1]), full)],
        out_specs=pl.BlockSpec((_RB, W_o2.shape[1]), row),
        out_shape=jax.ShapeDtypeStruct((n, W_o2.shape[1]), jnp.float32),
    )(s2, g2, degp, bg2.reshape(1, -1),
      W_o1, b_o1.reshape(1, -1), W_o2, b_o2.reshape(1, -1))

    return out


# speedup vs baseline: 1.1363x; 1.1319x over previous
import functools

import jax
import jax.numpy as jnp
from jax import lax
from jax.experimental import pallas as pl
from jax.experimental.pallas import tpu as pltpu
from jax.experimental.pallas import tpu_sc as plsc


def _leaky(x):
    return jnp.where(x >= 0, x, 0.01 * x)


_RB = 1000
_CH = 128
_NC = 2
_NS = 16


def _fusion_body(des_ref, tw_ref, np_ref, cp_ref,
                 wd_ref, bd_ref, wt_ref, bt_ref, wn_ref, bn_ref,
                 wc_ref, bc_ref, wi_ref, bi_ref, o_ref):
    d = _leaky(jnp.dot(des_ref[...], wd_ref[...],
                       preferred_element_type=jnp.float32) + bd_ref[...])
    t = _leaky(jnp.dot(tw_ref[...], wt_ref[...],
                       preferred_element_type=jnp.float32) + bt_ref[...])
    n = _leaky(jnp.dot(np_ref[...], wn_ref[...],
                       preferred_element_type=jnp.float32) + bn_ref[...])
    c = _leaky(jnp.dot(cp_ref[...], wc_ref[...],
                       preferred_element_type=jnp.float32) + bc_ref[...])
    x = jnp.concatenate([d, t, n, c], axis=1)
    o_ref[...] = _leaky(jnp.dot(x, wi_ref[...],
                                preferred_element_type=jnp.float32) + bi_ref[...])


def _fusion(des, tweet, num_prop, cat_prop,
            W_des, b_des, W_tw, b_tw, W_np, b_np, W_cp, b_cp, W_in, b_in):
    n = des.shape[0]
    q = W_des.shape[1]
    d = W_in.shape[1]
    row = lambda i: (i, 0)
    full = lambda i: (0, 0)
    return pl.pallas_call(
        _fusion_body,
        grid=(n // _RB,),
        in_specs=[
            pl.BlockSpec((_RB, des.shape[1]), row),
            pl.BlockSpec((_RB, tweet.shape[1]), row),
            pl.BlockSpec((_RB, num_prop.shape[1]), row),
            pl.BlockSpec((_RB, cat_prop.shape[1]), row),
            pl.BlockSpec(W_des.shape, full), pl.BlockSpec((1, q), full),
            pl.BlockSpec(W_tw.shape, full), pl.BlockSpec((1, q), full),
            pl.BlockSpec(W_np.shape, full), pl.BlockSpec((1, q), full),
            pl.BlockSpec(W_cp.shape, full), pl.BlockSpec((1, q), full),
            pl.BlockSpec(W_in.shape, full), pl.BlockSpec((1, d), full),
        ],
        out_specs=pl.BlockSpec((_RB, d), row),
        out_shape=jax.ShapeDtypeStruct((n, d), jnp.float32),
    )(des, tweet, num_prop, cat_prop,
      W_des, b_des.reshape(1, -1), W_tw, b_tw.reshape(1, -1),
      W_np, b_np.reshape(1, -1), W_cp, b_cp.reshape(1, -1),
      W_in, b_in.reshape(1, -1))


def _dinv_of(degp):
    return lax.rsqrt(degp[0] + degp[1] + 1.0)


def _pre_body(x_ref, w_ref, degp_ref, o_ref):
    h = jnp.dot(x_ref[...], w_ref[...], preferred_element_type=jnp.float32)
    g = _dinv_of(degp_ref[...]) * h
    d2 = g.shape[1] // 2
    o_ref[0] = g[:, :d2]
    o_ref[1] = g[:, d2:]


def _mid_body(s_ref, g_ref, degp_ref, b_ref, w_ref, o_ref):
    dinv = _dinv_of(degp_ref[...])
    s = jnp.concatenate([s_ref[0], s_ref[1]], axis=1)
    g = jnp.concatenate([g_ref[0], g_ref[1]], axis=1)
    x1 = dinv * (s + g) + b_ref[...]
    g2 = dinv * jnp.dot(x1, w_ref[...], preferred_element_type=jnp.float32)
    d2 = g2.shape[1] // 2
    o_ref[0] = g2[:, :d2]
    o_ref[1] = g2[:, d2:]


def _post_body(s_ref, g_ref, degp_ref, b_ref, w1_ref, b1_ref, w2_ref, b2_ref,
               o_ref):
    dinv = _dinv_of(degp_ref[...])
    s = jnp.concatenate([s_ref[0], s_ref[1]], axis=1)
    g = jnp.concatenate([g_ref[0], g_ref[1]], axis=1)
    x2 = dinv * (s + g) + b_ref[...]
    h = _leaky(jnp.dot(x2, w1_ref[...],
                       preferred_element_type=jnp.float32) + b1_ref[...])
    o_ref[...] = jnp.dot(h, w2_ref[...],
                         preferred_element_type=jnp.float32) + b2_ref[...]


def _sc_meshes():
    return plsc.VectorSubcoreMesh(core_axis_name="c", subcore_axis_name="s")


def _deg_call(dst2, nacc):
    rows = dst2.shape[0]
    rpt = rows // (_NC * _NS)
    grp = 4
    zcp = nacc // _NS // _CH

    @functools.partial(
        pl.kernel,
        out_type=jax.ShapeDtypeStruct((_NC, nacc), jnp.float32),
        mesh=_sc_meshes(),
        compiler_params=pltpu.CompilerParams(use_tc_tiling_on_sc=False),
        scratch_types=[
            pltpu.VMEM_SHARED((nacc,), jnp.float32),
            pltpu.VMEM((grp, _CH), jnp.int32),
            pltpu.VMEM((_CH,), jnp.float32),
            pltpu.VMEM((_CH,), jnp.float32),
        ],
    )
    def deg_kernel(dst2_hbm, degp_hbm, acc, idxb, onesb, zb):
        c = lax.axis_index("c")
        s = lax.axis_index("s")

        def fill(i, _):
            onesb[pl.ds(i * 16, 16)] = jnp.full((16,), 1.0, jnp.float32)
            zb[pl.ds(i * 16, 16)] = jnp.zeros((16,), jnp.float32)
            return 0
        lax.fori_loop(0, _CH // 16, fill, 0)

        def zero(i, _):
            pltpu.sync_copy(zb, acc.at[pl.ds((s * zcp + i) * _CH, _CH)])
            return 0
        lax.fori_loop(0, zcp, zero, 0)
        plsc.subcore_barrier()

        base = (c * _NS + s) * rpt

        def body(gi, _):
            row0 = base + gi * grp
            pltpu.sync_copy(dst2_hbm.at[pl.ds(row0, grp)], idxb)
            for j in range(grp):
                pltpu.sync_copy(onesb, acc.at[idxb.at[j]], add=True)
            return 0
        lax.fori_loop(0, rpt // grp, body, 0)
        plsc.subcore_barrier()

        def out(i, _):
            off = (s * zcp + i) * _CH
            pltpu.sync_copy(acc.at[pl.ds(off, _CH)],
                            degp_hbm.at[c, pl.ds(off, _CH)])
            return 0
        lax.fori_loop(0, zcp, out, 0)

    return deg_kernel(dst2)


def _agg_call(sr3, dst2, gflat, nacc):
    rows = dst2.shape[0]
    d2 = gflat.shape[1]
    rpt = rows // _NS
    grp = 4
    zcp = nacc // _NS // _CH

    @functools.partial(
        pl.kernel,
        out_type=jax.ShapeDtypeStruct((_NC, nacc, d2), jnp.float32),
        mesh=_sc_meshes(),
        compiler_params=pltpu.CompilerParams(use_tc_tiling_on_sc=False),
        scratch_types=[
            pltpu.VMEM_SHARED((nacc, d2), jnp.float32),
            pltpu.VMEM((grp, _CH), jnp.int32),
            pltpu.VMEM((grp, _CH), jnp.int32),
            pltpu.VMEM((grp, _CH, d2), jnp.float32),
            pltpu.SemaphoreType.DMA,
        ],
    )
    def agg_kernel(sr3_hbm, dst2_hbm, g_hbm, sout_hbm,
                   acc, sbuf, dbuf, rbuf, gsem):
        c = lax.axis_index("c")
        s = lax.axis_index("s")

        def zfill(i, _):
            for k in range(d2 // 16):
                rbuf[0, i, pl.ds(k * 16, 16)] = jnp.zeros((16,), jnp.float32)
            return 0
        lax.fori_loop(0, _CH, zfill, 0)

        def zero(i, _):
            pltpu.sync_copy(rbuf.at[0], acc.at[pl.ds((s * zcp + i) * _CH, _CH)])
            return 0
        lax.fori_loop(0, zcp, zero, 0)
        plsc.subcore_barrier()

        base = s * rpt

        def body(gi, _):
            row0 = base + gi * grp
            pltpu.sync_copy(sr3_hbm.at[c, pl.ds(row0, grp)], sbuf)
            pltpu.sync_copy(dst2_hbm.at[pl.ds(row0, grp)], dbuf)
            cps = [pltpu.async_copy(g_hbm.at[sbuf.at[j]], rbuf.at[j], gsem)
                   for j in range(grp)]
            for cp in cps:
                cp.wait()
            for j in range(grp):
                pltpu.sync_copy(rbuf.at[j], acc.at[dbuf.at[j]], add=True)
            return 0
        lax.fori_loop(0, rpt // grp, body, 0)
        plsc.subcore_barrier()

        def out(i, _):
            off = (s * zcp + i) * _CH
            pltpu.sync_copy(acc.at[pl.ds(off, _CH)],
                            sout_hbm.at[c, pl.ds(off, _CH)])
            return 0
        lax.fori_loop(0, zcp, out, 0)

    return agg_kernel(sr3, dst2, gflat)


def kernel(des, tweet, num_prop, cat_prop, edge_index, edge_type,
           W_des, b_des, W_tw, b_tw, W_np, b_np, W_cp, b_cp,
           W_in, b_in, Wg1, bg1, Wg2, bg2, W_o1, b_o1, W_o2, b_o2):
    n = des.shape[0]
    e = edge_index.shape[1]
    d = W_in.shape[1]
    d2 = d // 2

    unit = _CH * _NC * _NS
    pe = ((e + unit - 1) // unit) * unit
    pad = pe - e
    garbage = 1200
    nacc = ((n + garbage + _NS * _CH - 1) // (_NS * _CH)) * (_NS * _CH)
    src = jnp.concatenate(
        [edge_index[0], jnp.zeros((pad,), edge_index.dtype)])
    dst = jnp.concatenate(
        [edge_index[1],
         n + (jnp.arange(pad, dtype=edge_index.dtype) % garbage)])
    rows = pe // _CH
    sr3 = jnp.stack([src, src + n]).reshape(_NC, rows, _CH)
    dst2 = dst.reshape(rows, _CH)

    degp = _deg_call(dst2, nacc).reshape(_NC, nacc, 1)

    x = _fusion(des, tweet, num_prop, cat_prop,
                W_des, b_des, W_tw, b_tw, W_np, b_np, W_cp, b_cp, W_in, b_in)

    row = lambda i: (i, 0)
    full = lambda i: (0, 0)
    stk = lambda i: (0, i, 0)
    spec_half = pl.BlockSpec((_NC, _RB, d2), stk)
    spec_deg = pl.BlockSpec((_NC, _RB, 1), stk)
    grid = (n // _RB,)

    g1 = pl.pallas_call(
        _pre_body,
        grid=grid,
        in_specs=[pl.BlockSpec((_RB, d), row), pl.BlockSpec(Wg1.shape, full),
                  spec_deg],
        out_specs=spec_half,
        out_shape=jax.ShapeDtypeStruct((_NC, n, d2), jnp.float32),
    )(x, Wg1, degp)

    s1 = _agg_call(sr3, dst2, g1.reshape(_NC * n, d2), nacc)

    g2 = pl.pallas_call(
        _mid_body,
        grid=grid,
        in_specs=[spec_half, spec_half, spec_deg,
                  pl.BlockSpec((1, d), full), pl.BlockSpec(Wg2.shape, full)],
        out_specs=spec_half,
        out_shape=jax.ShapeDtypeStruct((_NC, n, d2), jnp.float32),
    )(s1, g1, degp, bg1.reshape(1, -1), Wg2)

    s2 = _agg_call(sr3, dst2, g2.reshape(_NC * n, d2), nacc)

    out = pl.pallas_call(
        _post_body,
        grid=grid,
        in_specs=[spec_half, spec_half, spec_deg,
                  pl.BlockSpec((1, d), full),
                  pl.BlockSpec(W_o1.shape, full), pl.BlockSpec((1, d), full),
                  pl.BlockSpec(W_o2.shape, full),
                  pl.BlockSpec((1, W_o2.shape[1]), full)],
        out_specs=pl.BlockSpec((_RB, W_o2.shape[1]), row),
        out_shape=jax.ShapeDtypeStruct((n, W_o2.shape[1]), jnp.float32),
    )(s2, g2, degp, bg2.reshape(1, -1),
      W_o1, b_o1.reshape(1, -1), W_o2, b_o2.reshape(1, -1))

    return out


# speedup vs baseline: 1.2168x; 1.0708x over previous
import functools

import jax
import jax.numpy as jnp
from jax import lax
from jax.experimental import pallas as pl
from jax.experimental.pallas import tpu as pltpu
from jax.experimental.pallas import tpu_sc as plsc


def _leaky(x):
    return jnp.where(x >= 0, x, 0.01 * x)


_RB = 1024
_CH = 128
_NC = 2
_NS = 16


def _fusion_body(des_ref, tw_ref, np_ref, cp_ref,
                 wd_ref, bd_ref, wt_ref, bt_ref, wn_ref, bn_ref,
                 wc_ref, bc_ref, wi_ref, bi_ref, wg_ref, degp_ref, o_ref):
    d = _leaky(jnp.dot(des_ref[...], wd_ref[...],
                       preferred_element_type=jnp.float32) + bd_ref[...])
    t = _leaky(jnp.dot(tw_ref[...], wt_ref[...],
                       preferred_element_type=jnp.float32) + bt_ref[...])
    n = _leaky(jnp.dot(np_ref[...], wn_ref[...],
                       preferred_element_type=jnp.float32) + bn_ref[...])
    c = _leaky(jnp.dot(cp_ref[...], wc_ref[...],
                       preferred_element_type=jnp.float32) + bc_ref[...])
    x = jnp.concatenate([d, t, n, c], axis=1)
    x = _leaky(jnp.dot(x, wi_ref[...],
                       preferred_element_type=jnp.float32) + bi_ref[...])
    h = jnp.dot(x, wg_ref[...], preferred_element_type=jnp.float32)
    g = _dinv_of(degp_ref[...]) * h
    d2 = g.shape[1] // 2
    o_ref[0] = g[:, :d2]
    o_ref[1] = g[:, d2:]


def _fusion(des, tweet, num_prop, cat_prop,
            W_des, b_des, W_tw, b_tw, W_np, b_np, W_cp, b_cp, W_in, b_in,
            Wg1, degp):
    n = des.shape[0]
    q = W_des.shape[1]
    d = W_in.shape[1]
    row = lambda i: (i, 0)
    full = lambda i: (0, 0)
    return pl.pallas_call(
        _fusion_body,
        grid=((n + _RB - 1) // _RB,),
        in_specs=[
            pl.BlockSpec((_RB, des.shape[1]), row),
            pl.BlockSpec((_RB, tweet.shape[1]), row),
            pl.BlockSpec((_RB, num_prop.shape[1]), row),
            pl.BlockSpec((_RB, cat_prop.shape[1]), row),
            pl.BlockSpec(W_des.shape, full), pl.BlockSpec((1, q), full),
            pl.BlockSpec(W_tw.shape, full), pl.BlockSpec((1, q), full),
            pl.BlockSpec(W_np.shape, full), pl.BlockSpec((1, q), full),
            pl.BlockSpec(W_cp.shape, full), pl.BlockSpec((1, q), full),
            pl.BlockSpec(W_in.shape, full), pl.BlockSpec((1, d), full),
            pl.BlockSpec(Wg1.shape, full),
            pl.BlockSpec((2, _RB), lambda i: (0, i)),
        ],
        out_specs=pl.BlockSpec((2, _RB, d // 2), lambda i: (0, i, 0)),
        out_shape=jax.ShapeDtypeStruct((2, n, d // 2), jnp.float32),
    )(des, tweet, num_prop, cat_prop,
      W_des, b_des.reshape(1, -1), W_tw, b_tw.reshape(1, -1),
      W_np, b_np.reshape(1, -1), W_cp, b_cp.reshape(1, -1),
      W_in, b_in.reshape(1, -1), Wg1, degp)


def _dinv_of(degp):
    return lax.rsqrt(degp[0] + degp[1] + 1.0)[:, None]


def _mid_body(s_ref, g_ref, degp_ref, b_ref, w_ref, o_ref):
    dinv = _dinv_of(degp_ref[...])
    s = jnp.concatenate([s_ref[0], s_ref[1]], axis=1)
    g = jnp.concatenate([g_ref[0], g_ref[1]], axis=1)
    x1 = dinv * (s + g) + b_ref[...]
    g2 = dinv * jnp.dot(x1, w_ref[...], preferred_element_type=jnp.float32)
    d2 = g2.shape[1] // 2
    o_ref[0] = g2[:, :d2]
    o_ref[1] = g2[:, d2:]


def _post_body(s_ref, g_ref, degp_ref, b_ref, w1_ref, b1_ref, w2_ref, b2_ref,
               o_ref):
    dinv = _dinv_of(degp_ref[...])
    s = jnp.concatenate([s_ref[0], s_ref[1]], axis=1)
    g = jnp.concatenate([g_ref[0], g_ref[1]], axis=1)
    x2 = dinv * (s + g) + b_ref[...]
    h = _leaky(jnp.dot(x2, w1_ref[...],
                       preferred_element_type=jnp.float32) + b1_ref[...])
    o_ref[...] = jnp.dot(h, w2_ref[...],
                         preferred_element_type=jnp.float32) + b2_ref[...]


def _sc_meshes():
    return plsc.VectorSubcoreMesh(core_axis_name="c", subcore_axis_name="s")


def _deg_call(dst2, nacc):
    rows = dst2.shape[0]
    rpt = rows // (_NC * _NS)
    grp = 4
    zcp = nacc // _NS // _CH

    @functools.partial(
        pl.kernel,
        out_type=jax.ShapeDtypeStruct((_NC, nacc), jnp.float32),
        mesh=_sc_meshes(),
        compiler_params=pltpu.CompilerParams(use_tc_tiling_on_sc=False),
        scratch_types=[
            pltpu.VMEM_SHARED((nacc,), jnp.float32),
            pltpu.VMEM((grp, _CH), jnp.int32),
            pltpu.VMEM((_CH,), jnp.float32),
            pltpu.VMEM((_CH,), jnp.float32),
        ],
    )
    def deg_kernel(dst2_hbm, degp_hbm, acc, idxb, onesb, zb):
        c = lax.axis_index("c")
        s = lax.axis_index("s")

        def fill(i, _):
            onesb[pl.ds(i * 16, 16)] = jnp.full((16,), 1.0, jnp.float32)
            zb[pl.ds(i * 16, 16)] = jnp.zeros((16,), jnp.float32)
            return 0
        lax.fori_loop(0, _CH // 16, fill, 0)

        def zero(i, _):
            pltpu.sync_copy(zb, acc.at[pl.ds((s * zcp + i) * _CH, _CH)])
            return 0
        lax.fori_loop(0, zcp, zero, 0)
        plsc.subcore_barrier()

        base = (c * _NS + s) * rpt

        def body(gi, _):
            row0 = base + gi * grp
            pltpu.sync_copy(dst2_hbm.at[pl.ds(row0, grp)], idxb)
            for j in range(grp):
                pltpu.sync_copy(onesb, acc.at[idxb.at[j]], add=True)
            return 0
        lax.fori_loop(0, rpt // grp, body, 0)
        plsc.subcore_barrier()

        def out(i, _):
            off = (s * zcp + i) * _CH
            pltpu.sync_copy(acc.at[pl.ds(off, _CH)],
                            degp_hbm.at[c, pl.ds(off, _CH)])
            return 0
        lax.fori_loop(0, zcp, out, 0)

    return deg_kernel(dst2)


def _agg_call(sr3, dst2, gflat, nacc):
    rows = dst2.shape[0]
    d2 = gflat.shape[1]
    rpt = rows // _NS
    grp = 4
    zcp = nacc // _NS // _CH

    @functools.partial(
        pl.kernel,
        out_type=jax.ShapeDtypeStruct((_NC, nacc, d2), jnp.float32),
        mesh=_sc_meshes(),
        compiler_params=pltpu.CompilerParams(use_tc_tiling_on_sc=False),
        scratch_types=[
            pltpu.VMEM_SHARED((nacc, d2), jnp.float32),
            pltpu.VMEM((grp, _CH), jnp.int32),
            pltpu.VMEM((grp, _CH), jnp.int32),
            pltpu.VMEM((grp, _CH, d2), jnp.float32),
            pltpu.SemaphoreType.DMA,
        ],
    )
    def agg_kernel(sr3_hbm, dst2_hbm, g_hbm, sout_hbm,
                   acc, sbuf, dbuf, rbuf, gsem):
        c = lax.axis_index("c")
        s = lax.axis_index("s")

        def zfill(i, _):
            for k in range(d2 // 16):
                rbuf[0, i, pl.ds(k * 16, 16)] = jnp.zeros((16,), jnp.float32)
            return 0
        lax.fori_loop(0, _CH, zfill, 0)

        def zero(i, _):
            pltpu.sync_copy(rbuf.at[0], acc.at[pl.ds((s * zcp + i) * _CH, _CH)])
            return 0
        lax.fori_loop(0, zcp, zero, 0)
        plsc.subcore_barrier()

        base = s * rpt

        def body(gi, _):
            row0 = base + gi * grp
            pltpu.sync_copy(sr3_hbm.at[c, pl.ds(row0, grp)], sbuf)
            pltpu.sync_copy(dst2_hbm.at[pl.ds(row0, grp)], dbuf)
            cps = [pltpu.async_copy(g_hbm.at[sbuf.at[j]], rbuf.at[j], gsem)
                   for j in range(grp)]
            for cp in cps:
                cp.wait()
            for j in range(grp):
                pltpu.sync_copy(rbuf.at[j], acc.at[dbuf.at[j]], add=True)
            return 0
        lax.fori_loop(0, rpt // grp, body, 0)
        plsc.subcore_barrier()

        def out(i, _):
            off = (s * zcp + i) * _CH
            pltpu.sync_copy(acc.at[pl.ds(off, _CH)],
                            sout_hbm.at[c, pl.ds(off, _CH)])
            return 0
        lax.fori_loop(0, zcp, out, 0)

    return agg_kernel(sr3, dst2, gflat)


def kernel(des, tweet, num_prop, cat_prop, edge_index, edge_type,
           W_des, b_des, W_tw, b_tw, W_np, b_np, W_cp, b_cp,
           W_in, b_in, Wg1, bg1, Wg2, bg2, W_o1, b_o1, W_o2, b_o2):
    n = des.shape[0]
    e = edge_index.shape[1]
    d = W_in.shape[1]
    d2 = d // 2

    unit = _CH * _NC * _NS
    pe = ((e + unit - 1) // unit) * unit
    pad = pe - e
    garbage = 1200
    nacc = ((n + garbage + _NS * _CH - 1) // (_NS * _CH)) * (_NS * _CH)
    src = jnp.concatenate(
        [edge_index[0], jnp.zeros((pad,), edge_index.dtype)])
    dst = jnp.concatenate(
        [edge_index[1],
         n + (jnp.arange(pad, dtype=edge_index.dtype) % garbage)])
    rows = pe // _CH
    sr3 = jnp.stack([src, src + n]).reshape(_NC, rows, _CH)
    dst2 = dst.reshape(rows, _CH)

    degp = _deg_call(dst2, nacc)

    g1 = _fusion(des, tweet, num_prop, cat_prop,
                 W_des, b_des, W_tw, b_tw, W_np, b_np, W_cp, b_cp,
                 W_in, b_in, Wg1, degp)

    row = lambda i: (i, 0)
    full = lambda i: (0, 0)
    stk = lambda i: (0, i, 0)
    spec_half = pl.BlockSpec((_NC, _RB, d2), stk)
    spec_deg = pl.BlockSpec((_NC, _RB), lambda i: (0, i))
    grid = ((n + _RB - 1) // _RB,)

    s1 = _agg_call(sr3, dst2, g1.reshape(_NC * n, d2), nacc)

    g2 = pl.pallas_call(
        _mid_body,
        grid=grid,
        in_specs=[spec_half, spec_half, spec_deg,
                  pl.BlockSpec((1, d), full), pl.BlockSpec(Wg2.shape, full)],
        out_specs=spec_half,
        out_shape=jax.ShapeDtypeStruct((_NC, n, d2), jnp.float32),
    )(s1, g1, degp, bg1.reshape(1, -1), Wg2)

    s2 = _agg_call(sr3, dst2, g2.reshape(_NC * n, d2), nacc)

    out = pl.pallas_call(
        _post_body,
        grid=grid,
        in_specs=[spec_half, spec_half, spec_deg,
                  pl.BlockSpec((1, d), full),
                  pl.BlockSpec(W_o1.shape, full), pl.BlockSpec((1, d), full),
                  pl.BlockSpec(W_o2.shape, full),
                  pl.BlockSpec((1, W_o2.shape[1]), full)],
        out_specs=pl.BlockSpec((_RB, W_o2.shape[1]), row),
        out_shape=jax.ShapeDtypeStruct((n, W_o2.shape[1]), jnp.float32),
    )(s2, g2, degp, bg2.reshape(1, -1),
      W_o1, b_o1.reshape(1, -1), W_o2, b_o2.reshape(1, -1))

    return out


# speedup vs baseline: 1.2555x; 1.0318x over previous
import functools

import jax
import jax.numpy as jnp
from jax import lax
from jax.experimental import pallas as pl
from jax.experimental.pallas import tpu as pltpu
from jax.experimental.pallas import tpu_sc as plsc


def _leaky(x):
    return jnp.where(x >= 0, x, 0.01 * x)


_RB = 1024
_CH = 128
_NC = 2
_NS = 16


def _fusion_body(des_ref, tw_ref, np_ref, cp_ref,
                 wd_ref, bd_ref, wt_ref, bt_ref, wn_ref, bn_ref,
                 wc_ref, bc_ref, wi_ref, bi_ref, wg_ref, degp_ref, o_ref):
    d = _leaky(jnp.dot(des_ref[...], wd_ref[...],
                       preferred_element_type=jnp.float32) + bd_ref[...])
    t = _leaky(jnp.dot(tw_ref[...], wt_ref[...],
                       preferred_element_type=jnp.float32) + bt_ref[...])
    n = _leaky(jnp.dot(np_ref[...], wn_ref[...],
                       preferred_element_type=jnp.float32) + bn_ref[...])
    c = _leaky(jnp.dot(cp_ref[...], wc_ref[...],
                       preferred_element_type=jnp.float32) + bc_ref[...])
    x = jnp.concatenate([d, t, n, c], axis=1)
    x = _leaky(jnp.dot(x, wi_ref[...],
                       preferred_element_type=jnp.float32) + bi_ref[...])
    h = jnp.dot(x, wg_ref[...], preferred_element_type=jnp.float32)
    g = _dinv_of(degp_ref[...]) * h
    d2 = g.shape[1] // 2
    o_ref[0] = g[:, :d2]
    o_ref[1] = g[:, d2:]


def _fusion(des, tweet, num_prop, cat_prop,
            W_des, b_des, W_tw, b_tw, W_np, b_np, W_cp, b_cp, W_in, b_in,
            Wg1, degp):
    n = des.shape[0]
    q = W_des.shape[1]
    d = W_in.shape[1]
    row = lambda i: (i, 0)
    full = lambda i: (0, 0)
    return pl.pallas_call(
        _fusion_body,
        grid=((n + _RB - 1) // _RB,),
        in_specs=[
            pl.BlockSpec((_RB, des.shape[1]), row),
            pl.BlockSpec((_RB, tweet.shape[1]), row),
            pl.BlockSpec((_RB, num_prop.shape[1]), row),
            pl.BlockSpec((_RB, cat_prop.shape[1]), row),
            pl.BlockSpec(W_des.shape, full), pl.BlockSpec((1, q), full),
            pl.BlockSpec(W_tw.shape, full), pl.BlockSpec((1, q), full),
            pl.BlockSpec(W_np.shape, full), pl.BlockSpec((1, q), full),
            pl.BlockSpec(W_cp.shape, full), pl.BlockSpec((1, q), full),
            pl.BlockSpec(W_in.shape, full), pl.BlockSpec((1, d), full),
            pl.BlockSpec(Wg1.shape, full),
            pl.BlockSpec((2, _RB), lambda i: (0, i)),
        ],
        out_specs=pl.BlockSpec((2, _RB, d // 2), lambda i: (0, i, 0)),
        out_shape=jax.ShapeDtypeStruct((2, n, d // 2), jnp.float32),
    )(des, tweet, num_prop, cat_prop,
      W_des, b_des.reshape(1, -1), W_tw, b_tw.reshape(1, -1),
      W_np, b_np.reshape(1, -1), W_cp, b_cp.reshape(1, -1),
      W_in, b_in.reshape(1, -1), Wg1, degp)


def _dinv_of(degp):
    return lax.rsqrt(degp[0] + degp[1] + 1.0)[:, None]


def _mid_body(s_ref, g_ref, degp_ref, b_ref, w_ref, o_ref):
    dinv = _dinv_of(degp_ref[...])
    s = jnp.concatenate([s_ref[0], s_ref[1]], axis=1)
    g = jnp.concatenate([g_ref[0], g_ref[1]], axis=1)
    x1 = dinv * (s + g) + b_ref[...]
    g2 = dinv * jnp.dot(x1, w_ref[...], preferred_element_type=jnp.float32)
    d2 = g2.shape[1] // 2
    o_ref[0] = g2[:, :d2]
    o_ref[1] = g2[:, d2:]


def _post_body(s_ref, g_ref, degp_ref, b_ref, w1_ref, b1_ref, w2_ref, b2_ref,
               o_ref):
    dinv = _dinv_of(degp_ref[...])
    s = jnp.concatenate([s_ref[0], s_ref[1]], axis=1)
    g = jnp.concatenate([g_ref[0], g_ref[1]], axis=1)
    x2 = dinv * (s + g) + b_ref[...]
    h = _leaky(jnp.dot(x2, w1_ref[...],
                       preferred_element_type=jnp.float32) + b1_ref[...])
    o_ref[...] = jnp.dot(h, w2_ref[...],
                         preferred_element_type=jnp.float32) + b2_ref[...]


def _sc_meshes():
    return plsc.VectorSubcoreMesh(core_axis_name="c", subcore_axis_name="s")


def _deg_call(dstr, nacc):
    rows = dstr.shape[0]
    rpt = (rows + _NC * _NS - 1) // (_NC * _NS)
    grp = 4
    zcp = nacc // _NS // _CH

    @functools.partial(
        pl.kernel,
        out_type=jax.ShapeDtypeStruct((_NC, nacc), jnp.float32),
        mesh=_sc_meshes(),
        compiler_params=pltpu.CompilerParams(use_tc_tiling_on_sc=False),
        scratch_types=[
            pltpu.VMEM_SHARED((nacc,), jnp.float32),
            pltpu.VMEM((grp, _CH), jnp.int32),
            pltpu.VMEM((_CH,), jnp.float32),
            pltpu.VMEM((_CH,), jnp.float32),
        ],
    )
    def deg_kernel(dstr_hbm, degp_hbm, acc, idxb, onesb, zb):
        c = lax.axis_index("c")
        s = lax.axis_index("s")

        def fill(i, _):
            onesb[pl.ds(i * 16, 16)] = jnp.full((16,), 1.0, jnp.float32)
            zb[pl.ds(i * 16, 16)] = jnp.zeros((16,), jnp.float32)
            return 0
        lax.fori_loop(0, _CH // 16, fill, 0)

        def zero(i, _):
            pltpu.sync_copy(zb, acc.at[pl.ds((s * zcp + i) * _CH, _CH)])
            return 0
        lax.fori_loop(0, zcp, zero, 0)
        plsc.subcore_barrier()

        base = (c * _NS + s) * rpt

        def body(gi, _):
            row0 = base + gi * grp

            @pl.when(row0 + grp <= rows)
            def _full():
                pltpu.sync_copy(dstr_hbm.at[pl.ds(row0, grp)], idxb)
                for j in range(grp):
                    pltpu.sync_copy(onesb, acc.at[idxb.at[j]], add=True)

            @pl.when(jnp.logical_and(row0 < rows, row0 + grp > rows))
            def _tail():
                def one(j, _):
                    pltpu.sync_copy(dstr_hbm.at[pl.ds(row0 + j, 1)],
                                    idxb.at[pl.ds(0, 1)])
                    pltpu.sync_copy(onesb, acc.at[idxb.at[0]], add=True)
                    return 0
                lax.fori_loop(0, rows - row0, one, 0)
            return 0
        lax.fori_loop(0, (rpt + grp - 1) // grp, body, 0)
        plsc.subcore_barrier()

        def out(i, _):
            off = (s * zcp + i) * _CH
            pltpu.sync_copy(acc.at[pl.ds(off, _CH)],
                            degp_hbm.at[c, pl.ds(off, _CH)])
            return 0
        lax.fori_loop(0, zcp, out, 0)

    return deg_kernel(dstr)


def _agg_call(sr3, dst2, gflat, nacc):
    rows = dst2.shape[0]
    d2 = gflat.shape[1]
    rpt = rows // _NS
    grp = 4
    zcp = nacc // _NS // _CH

    @functools.partial(
        pl.kernel,
        out_type=jax.ShapeDtypeStruct((_NC, nacc, d2), jnp.float32),
        mesh=_sc_meshes(),
        compiler_params=pltpu.CompilerParams(use_tc_tiling_on_sc=False),
        scratch_types=[
            pltpu.VMEM_SHARED((nacc, d2), jnp.float32),
            pltpu.VMEM((grp, _CH), jnp.int32),
            pltpu.VMEM((grp, _CH), jnp.int32),
            pltpu.VMEM((grp, _CH, d2), jnp.float32),
            pltpu.SemaphoreType.DMA,
            pltpu.SemaphoreType.DMA,
        ],
    )
    def agg_kernel(sr3_hbm, dst2_hbm, g_hbm, sout_hbm,
                   acc, sbuf, dbuf, rbuf, gsem, ssem):
        c = lax.axis_index("c")
        s = lax.axis_index("s")

        def zfill(i, _):
            for k in range(d2 // 16):
                rbuf[0, i, pl.ds(k * 16, 16)] = jnp.zeros((16,), jnp.float32)
            return 0
        lax.fori_loop(0, _CH, zfill, 0)

        def zero(i, _):
            zps = [pltpu.async_copy(
                rbuf.at[0], acc.at[pl.ds(((s * 5 + i) * 5 + k) * _CH, _CH)],
                ssem) for k in range(zcp // 5)]
            for zp in zps:
                zp.wait()
            return 0
        lax.fori_loop(0, 5, zero, 0)
        plsc.subcore_barrier()

        base = s * rpt

        def body(gi, _):
            row0 = base + gi * grp
            pltpu.sync_copy(sr3_hbm.at[c, pl.ds(row0, grp)], sbuf)
            pltpu.sync_copy(dst2_hbm.at[pl.ds(row0, grp)], dbuf)
            cps = [pltpu.async_copy(g_hbm.at[sbuf.at[j]], rbuf.at[j], gsem)
                   for j in range(grp)]
            for cp in cps:
                cp.wait()
            sps = [pltpu.async_copy(rbuf.at[j], acc.at[dbuf.at[j]], ssem,
                                    add=True) for j in range(grp)]
            for sp in sps:
                sp.wait()
            return 0
        lax.fori_loop(0, rpt // grp, body, 0)
        plsc.subcore_barrier()

        def out(i, _):
            off = (s * zcp + i) * _CH
            pltpu.sync_copy(acc.at[pl.ds(off, _CH)],
                            sout_hbm.at[c, pl.ds(off, _CH)])
            return 0
        lax.fori_loop(0, zcp, out, 0)

    return agg_kernel(sr3, dst2, gflat)


def kernel(des, tweet, num_prop, cat_prop, edge_index, edge_type,
           W_des, b_des, W_tw, b_tw, W_np, b_np, W_cp, b_cp,
           W_in, b_in, Wg1, bg1, Wg2, bg2, W_o1, b_o1, W_o2, b_o2):
    n = des.shape[0]
    e = edge_index.shape[1]
    d = W_in.shape[1]
    d2 = d // 2

    unit = _CH * _NC * _NS
    pe = ((e + unit - 1) // unit) * unit
    pad = pe - e
    garbage = 1200
    nacc = ((n + garbage + _NS * _CH - 1) // (_NS * _CH)) * (_NS * _CH)
    src = jnp.concatenate(
        [edge_index[0], jnp.zeros((pad,), edge_index.dtype)])
    dst = jnp.concatenate(
        [edge_index[1],
         n + (jnp.arange(pad, dtype=edge_index.dtype) % garbage)])
    rows = pe // _CH
    sr3 = jnp.stack([src, src + n]).reshape(_NC, rows, _CH)
    dst2 = dst.reshape(rows, _CH)

    degp = _deg_call(edge_index[1].reshape(e // _CH, _CH), nacc)

    g1 = _fusion(des, tweet, num_prop, cat_prop,
                 W_des, b_des, W_tw, b_tw, W_np, b_np, W_cp, b_cp,
                 W_in, b_in, Wg1, degp)

    row = lambda i: (i, 0)
    full = lambda i: (0, 0)
    stk = lambda i: (0, i, 0)
    spec_half = pl.BlockSpec((_NC, _RB, d2), stk)
    spec_deg = pl.BlockSpec((_NC, _RB), lambda i: (0, i))
    grid = ((n + _RB - 1) // _RB,)

    s1 = _agg_call(sr3, dst2, g1.reshape(_NC * n, d2), nacc)

    g2 = pl.pallas_call(
        _mid_body,
        grid=grid,
        in_specs=[spec_half, spec_half, spec_deg,
                  pl.BlockSpec((1, d), full), pl.BlockSpec(Wg2.shape, full)],
        out_specs=spec_half,
        out_shape=jax.ShapeDtypeStruct((_NC, n, d2), jnp.float32),
    )(s1, g1, degp, bg1.reshape(1, -1), Wg2)

    s2 = _agg_call(sr3, dst2, g2.reshape(_NC * n, d2), nacc)

    out = pl.pallas_call(
        _post_body,
        grid=grid,
        in_specs=[spec_half, spec_half, spec_deg,
                  pl.BlockSpec((1, d), full),
                  pl.BlockSpec(W_o1.shape, full), pl.BlockSpec((1, d), full),
                  pl.BlockSpec(W_o2.shape, full),
                  pl.BlockSpec((1, W_o2.shape[1]), full)],
        out_specs=pl.BlockSpec((_RB, W_o2.shape[1]), row),
        out_shape=jax.ShapeDtypeStruct((n, W_o2.shape[1]), jnp.float32),
    )(s2, g2, degp, bg2.reshape(1, -1),
      W_o1, b_o1.reshape(1, -1), W_o2, b_o2.reshape(1, -1))

    return out


# speedup vs baseline: 1.3468x; 1.0728x over previous
import functools

import jax
import jax.numpy as jnp
from jax import lax
from jax.experimental import pallas as pl
from jax.experimental.pallas import tpu as pltpu
from jax.experimental.pallas import tpu_sc as plsc


def _leaky(x):
    return jnp.where(x >= 0, x, 0.01 * x)


_RB = 1024
_CH = 128
_NC = 2
_NS = 16


def _fusion_body(des_ref, tw_ref, np_ref, cp_ref,
                 wd_ref, bd_ref, wt_ref, bt_ref, wn_ref, bn_ref,
                 wc_ref, bc_ref, wi_ref, bi_ref, wg_ref, degp_ref, o_ref):
    d = _leaky(jnp.dot(des_ref[...], wd_ref[...],
                       preferred_element_type=jnp.float32) + bd_ref[...])
    t = _leaky(jnp.dot(tw_ref[...], wt_ref[...],
                       preferred_element_type=jnp.float32) + bt_ref[...])
    n = _leaky(jnp.dot(np_ref[...], wn_ref[...],
                       preferred_element_type=jnp.float32) + bn_ref[...])
    c = _leaky(jnp.dot(cp_ref[...], wc_ref[...],
                       preferred_element_type=jnp.float32) + bc_ref[...])
    x = jnp.concatenate([d, t, n, c], axis=1)
    x = _leaky(jnp.dot(x, wi_ref[...],
                       preferred_element_type=jnp.float32) + bi_ref[...])
    h = jnp.dot(x, wg_ref[...], preferred_element_type=jnp.float32)
    g = _dinv_of(degp_ref[...]) * h
    d2 = g.shape[1] // 2
    o_ref[0] = g[:, :d2]
    o_ref[1] = g[:, d2:]


def _fusion(des, tweet, num_prop, cat_prop,
            W_des, b_des, W_tw, b_tw, W_np, b_np, W_cp, b_cp, W_in, b_in,
            Wg1, degp):
    n = des.shape[0]
    q = W_des.shape[1]
    d = W_in.shape[1]
    row = lambda i: (i, 0)
    full = lambda i: (0, 0)
    return pl.pallas_call(
        _fusion_body,
        grid=((n + _RB - 1) // _RB,),
        in_specs=[
            pl.BlockSpec((_RB, des.shape[1]), row),
            pl.BlockSpec((_RB, tweet.shape[1]), row),
            pl.BlockSpec((_RB, num_prop.shape[1]), row),
            pl.BlockSpec((_RB, cat_prop.shape[1]), row),
            pl.BlockSpec(W_des.shape, full), pl.BlockSpec((1, q), full),
            pl.BlockSpec(W_tw.shape, full), pl.BlockSpec((1, q), full),
            pl.BlockSpec(W_np.shape, full), pl.BlockSpec((1, q), full),
            pl.BlockSpec(W_cp.shape, full), pl.BlockSpec((1, q), full),
            pl.BlockSpec(W_in.shape, full), pl.BlockSpec((1, d), full),
            pl.BlockSpec(Wg1.shape, full),
            pl.BlockSpec((2, _RB), lambda i: (0, i)),
        ],
        out_specs=pl.BlockSpec((2, _RB, d // 2), lambda i: (0, i, 0)),
        out_shape=jax.ShapeDtypeStruct((2, n, d // 2), jnp.float32),
    )(des, tweet, num_prop, cat_prop,
      W_des, b_des.reshape(1, -1), W_tw, b_tw.reshape(1, -1),
      W_np, b_np.reshape(1, -1), W_cp, b_cp.reshape(1, -1),
      W_in, b_in.reshape(1, -1), Wg1, degp)


def _dinv_of(degp):
    return lax.rsqrt(degp[0] + degp[1] + 1.0)[:, None]


def _mid_body(s_ref, g_ref, degp_ref, b_ref, w_ref, o_ref):
    d2 = g_ref.shape[2]
    dinv = _dinv_of(degp_ref[...])
    s = jnp.concatenate([s_ref[0, :, :d2], s_ref[1, :, :d2]], axis=1)
    g = jnp.concatenate([g_ref[0], g_ref[1]], axis=1)
    x1 = dinv * (s + g) + b_ref[...]
    g2 = dinv * jnp.dot(x1, w_ref[...], preferred_element_type=jnp.float32)
    d2 = g2.shape[1] // 2
    o_ref[0] = g2[:, :d2]
    o_ref[1] = g2[:, d2:]


def _post_body(s_ref, g_ref, degp_ref, b_ref, w1_ref, b1_ref, w2_ref, b2_ref,
               o_ref):
    d2 = g_ref.shape[2]
    dinv = _dinv_of(degp_ref[...])
    s = jnp.concatenate([s_ref[0, :, :d2], s_ref[1, :, :d2]], axis=1)
    g = jnp.concatenate([g_ref[0], g_ref[1]], axis=1)
    x2 = dinv * (s + g) + b_ref[...]
    h = _leaky(jnp.dot(x2, w1_ref[...],
                       preferred_element_type=jnp.float32) + b1_ref[...])
    o_ref[...] = jnp.dot(h, w2_ref[...],
                         preferred_element_type=jnp.float32) + b2_ref[...]


def _sc_meshes():
    return plsc.VectorSubcoreMesh(core_axis_name="c", subcore_axis_name="s")


def _deg_call(ei3, nacc):
    rows = ei3.shape[1]
    rpt = (rows + _NC * _NS - 1) // (_NC * _NS)
    grp = 4
    zcp = nacc // _NS // _CH

    @functools.partial(
        pl.kernel,
        out_type=jax.ShapeDtypeStruct((_NC, nacc), jnp.float32),
        mesh=_sc_meshes(),
        compiler_params=pltpu.CompilerParams(use_tc_tiling_on_sc=False),
        scratch_types=[
            pltpu.VMEM_SHARED((nacc,), jnp.float32),
            pltpu.VMEM((grp, _CH), jnp.int32),
            pltpu.VMEM((_CH,), jnp.float32),
            pltpu.VMEM((_CH,), jnp.float32),
        ],
    )
    def deg_kernel(ei3_hbm, degp_hbm, acc, idxb, onesb, zb):
        dstr_hbm = ei3_hbm.at[1]
        c = lax.axis_index("c")
        s = lax.axis_index("s")

        def fill(i, _):
            onesb[pl.ds(i * 16, 16)] = jnp.full((16,), 1.0, jnp.float32)
            zb[pl.ds(i * 16, 16)] = jnp.zeros((16,), jnp.float32)
            return 0
        lax.fori_loop(0, _CH // 16, fill, 0)

        def zero(i, _):
            pltpu.sync_copy(zb, acc.at[pl.ds((s * zcp + i) * _CH, _CH)])
            return 0
        lax.fori_loop(0, zcp, zero, 0)
        plsc.subcore_barrier()

        base = (c * _NS + s) * rpt

        def body(gi, _):
            row0 = base + gi * grp

            @pl.when(row0 + grp <= rows)
            def _full():
                pltpu.sync_copy(dstr_hbm.at[pl.ds(row0, grp)], idxb)
                for j in range(grp):
                    pltpu.sync_copy(onesb, acc.at[idxb.at[j]], add=True)

            @pl.when(jnp.logical_and(row0 < rows, row0 + grp > rows))
            def _tail():
                def one(j, _):
                    pltpu.sync_copy(dstr_hbm.at[pl.ds(row0 + j, 1)],
                                    idxb.at[pl.ds(0, 1)])
                    pltpu.sync_copy(onesb, acc.at[idxb.at[0]], add=True)
                    return 0
                lax.fori_loop(0, rows - row0, one, 0)
            return 0
        lax.fori_loop(0, (rpt + grp - 1) // grp, body, 0)
        plsc.subcore_barrier()

        def out(i, _):
            off = (s * zcp + i) * _CH
            pltpu.sync_copy(acc.at[pl.ds(off, _CH)],
                            degp_hbm.at[c, pl.ds(off, _CH)])
            return 0
        lax.fori_loop(0, zcp, out, 0)

    return deg_kernel(ei3)


def _agg_call(sr3, dst2, gflat, nacc):
    rows = dst2.shape[0]
    d2 = gflat.shape[1]
    rpt = rows // _NS
    grp = 4
    zcp = nacc // _NS // _CH

    @functools.partial(
        pl.kernel,
        out_type=jax.ShapeDtypeStruct((_NC, nacc, _CH), jnp.float32),
        mesh=_sc_meshes(),
        compiler_params=pltpu.CompilerParams(use_tc_tiling_on_sc=False),
        scratch_types=[
            pltpu.VMEM_SHARED((nacc, d2), jnp.float32),
            pltpu.VMEM((grp, _CH), jnp.int32),
            pltpu.VMEM((grp, _CH), jnp.int32),
            pltpu.VMEM((grp, _CH, d2), jnp.float32),
            pltpu.SemaphoreType.DMA,
            pltpu.SemaphoreType.DMA,
        ],
    )
    def agg_kernel(sr3_hbm, dst2_hbm, g_hbm, sout_hbm,
                   acc, sbuf, dbuf, rbuf, gsem, ssem):
        c = lax.axis_index("c")
        s = lax.axis_index("s")

        def zfill(i, _):
            for k in range(d2 // 16):
                rbuf[0, i, pl.ds(k * 16, 16)] = jnp.zeros((16,), jnp.float32)
            return 0
        lax.fori_loop(0, _CH, zfill, 0)

        def zero(i, _):
            zps = [pltpu.async_copy(
                rbuf.at[0], acc.at[pl.ds(((s * 5 + i) * 5 + k) * _CH, _CH)],
                ssem) for k in range(zcp // 5)]
            for zp in zps:
                zp.wait()
            return 0
        lax.fori_loop(0, 5, zero, 0)
        plsc.subcore_barrier()

        base = s * rpt

        def body(gi, _):
            row0 = base + gi * grp
            pltpu.sync_copy(sr3_hbm.at[c, pl.ds(row0, grp)], sbuf)
            pltpu.sync_copy(dst2_hbm.at[pl.ds(row0, grp)], dbuf)
            cps = [pltpu.async_copy(g_hbm.at[sbuf.at[j]], rbuf.at[j], gsem)
                   for j in range(grp)]
            for cp in cps:
                cp.wait()
            sps = [pltpu.async_copy(rbuf.at[j], acc.at[dbuf.at[j]], ssem,
                                    add=True) for j in range(grp)]
            for sp in sps:
                sp.wait()
            return 0
        lax.fori_loop(0, rpt // grp, body, 0)
        plsc.subcore_barrier()

        def out(i, _):
            off = (s * zcp + i) * _CH
            pltpu.sync_copy(acc.at[pl.ds(off, _CH)],
                            sout_hbm.at[c, pl.ds(off, _CH), pl.ds(0, d2)])
            return 0
        lax.fori_loop(0, zcp, out, 0)

    return agg_kernel(sr3, dst2, gflat)


def kernel(des, tweet, num_prop, cat_prop, edge_index, edge_type,
           W_des, b_des, W_tw, b_tw, W_np, b_np, W_cp, b_cp,
           W_in, b_in, Wg1, bg1, Wg2, bg2, W_o1, b_o1, W_o2, b_o2):
    n = des.shape[0]
    e = edge_index.shape[1]
    d = W_in.shape[1]
    d2 = d // 2

    unit = _CH * _NC * _NS
    pe = ((e + unit - 1) // unit) * unit
    pad = pe - e
    garbage = 1200
    nacc = ((n + garbage + _NS * _CH - 1) // (_NS * _CH)) * (_NS * _CH)
    src = jnp.concatenate(
        [edge_index[0], jnp.zeros((pad,), edge_index.dtype)])
    dst = jnp.concatenate(
        [edge_index[1],
         n + (jnp.arange(pad, dtype=edge_index.dtype) % garbage)])
    rows = pe // _CH
    sr3 = jnp.stack([src, src + n]).reshape(_NC, rows, _CH)
    dst2 = dst.reshape(rows, _CH)

    degp = _deg_call(edge_index.reshape(2, e // _CH, _CH), nacc)

    g1 = _fusion(des, tweet, num_prop, cat_prop,
                 W_des, b_des, W_tw, b_tw, W_np, b_np, W_cp, b_cp,
                 W_in, b_in, Wg1, degp)

    row = lambda i: (i, 0)
    full = lambda i: (0, 0)
    stk = lambda i: (0, i, 0)
    spec_half = pl.BlockSpec((_NC, _RB, d2), stk)
    spec_s = pl.BlockSpec((_NC, _RB, _CH), stk)
    spec_deg = pl.BlockSpec((_NC, _RB), lambda i: (0, i))
    grid = ((n + _RB - 1) // _RB,)

    s1 = _agg_call(sr3, dst2, g1.reshape(_NC * n, d2), nacc)

    g2 = pl.pallas_call(
        _mid_body,
        grid=grid,
        in_specs=[spec_s, spec_half, spec_deg,
                  pl.BlockSpec((1, d), full), pl.BlockSpec(Wg2.shape, full)],
        out_specs=spec_half,
        out_shape=jax.ShapeDtypeStruct((_NC, n, d2), jnp.float32),
    )(s1, g1, degp, bg1.reshape(1, -1), Wg2)

    s2 = _agg_call(sr3, dst2, g2.reshape(_NC * n, d2), nacc)

    out = pl.pallas_call(
        _post_body,
        grid=grid,
        in_specs=[spec_s, spec_half, spec_deg,
                  pl.BlockSpec((1, d), full),
                  pl.BlockSpec(W_o1.shape, full), pl.BlockSpec((1, d), full),
                  pl.BlockSpec(W_o2.shape, full),
                  pl.BlockSpec((1, W_o2.shape[1]), full)],
        out_specs=pl.BlockSpec((_RB, W_o2.shape[1]), row),
        out_shape=jax.ShapeDtypeStruct((n, W_o2.shape[1]), jnp.float32),
    )(s2, g2, degp, bg2.reshape(1, -1),
      W_o1, b_o1.reshape(1, -1), W_o2, b_o2.reshape(1, -1))

    return out
